# baseline (speedup 1.0000x reference)
import functools

import jax
import jax.numpy as jnp
from jax import lax
from jax.experimental import pallas as pl
from jax.experimental.pallas import tpu as pltpu
from jax.experimental.pallas import tpu_sc as plsc

VOCAB = 100000
D = 64
B = 4096
CTX = 20

NW = 32
BPW = B // NW
IPW = BPW * CTX
CHUNKS = 8
ROWS_PER_CHUNK = BPW // CHUNKS
IDX_PER_CHUNK = ROWS_PER_CHUNK * CTX
SUBS = ((0, 128), (128, 128), (256, 64))
LANES = 16
WIDE = 2 * D

VT = 1024


def _mean_pool_sc(idx2, emb2):
    mesh = plsc.VectorSubcoreMesh(core_axis_name="c", subcore_axis_name="s")

    @functools.partial(
        pl.kernel,
        mesh=mesh,
        compiler_params=pltpu.CompilerParams(use_tc_tiling_on_sc=True),
        out_type=jax.ShapeDtypeStruct((NW, BPW // 2, WIDE), jnp.float32),
        scratch_types=[
            pltpu.VMEM((IPW,), jnp.int32),
            pltpu.VMEM((IPW,), jnp.int32),
            pltpu.VMEM((IPW + LANES,), jnp.float32),
            pltpu.VMEM((2, IDX_PER_CHUNK, WIDE), jnp.float32),
            pltpu.VMEM((BPW // 2, WIDE), jnp.float32),
            pltpu.SemaphoreType.DMA,
            pltpu.SemaphoreType.DMA,
        ],
    )
    def k(idx_hbm, emb_hbm, out_hbm, idx_v, idxw_v, maskf_v, rows_v, acc_v,
          sem0, sem1):
        wid = lax.axis_index("s") * 2 + lax.axis_index("c")
        sems = (sem0, sem1)
        zidx = jnp.zeros((LANES,), jnp.int32)
        pltpu.sync_copy(idx_hbm.at[wid], idx_v)

        def shift(i, carry):
            v = idx_v[pl.ds(i * LANES, LANES)]
            idxw_v[pl.ds(i * LANES, LANES)] = v >> 1
            maskf_v[pl.ds(i * LANES, LANES)] = (v & 1).astype(jnp.float32)
            return carry

        lax.fori_loop(0, IPW // LANES, shift, 0)

        def fire(c):
            return [
                pltpu.async_copy(
                    emb_hbm.at[idxw_v.at[pl.ds(c * IDX_PER_CHUNK + o, n)]],
                    rows_v.at[c % 2, pl.ds(o, n)],
                    sems[c % 2],
                )
                for o, n in SUBS
            ]

        pending = fire(0)
        for c in range(CHUNKS):
            for h in pending:
                h.wait()
            if c + 1 < CHUNKS:
                pending = fire(c + 1)

            def body(r, carry, c=c):
                base = r * CTX
                buf = c % 2
                masks = [
                    jnp.take(
                        maskf_v[pl.ds(c * IDX_PER_CHUNK + base + kk, LANES)],
                        zidx)
                    for kk in range(CTX)
                ]
                for g in range(D // LANES):
                    s = None
                    for kk in range(CTX):
                        lo = rows_v[buf, base + kk, pl.ds(g * LANES, LANES)]
                        hi = rows_v[buf, base + kk,
                                    pl.ds(D + g * LANES, LANES)]
                        v = lo + masks[kk] * (hi - lo)
                        s = v if s is None else s + v
                    row = c * ROWS_PER_CHUNK + r
                    acc_v[row // 2, pl.ds((row % 2) * D + g * LANES, LANES)] = (
                        s * (1.0 / CTX)
                    )
                return carry

            lax.fori_loop(0, ROWS_PER_CHUNK, body, 0)
        pltpu.sync_copy(acc_v, out_hbm.at[wid])

    return k(idx2, emb2)


def _logits_tc(wt_aug, mean_aug):
    def body(w_ref, mean_ref, out_ref):
        out_ref[...] = lax.dot_general(
            w_ref[...], mean_ref[...], (((0,), (1,)), ((), ())),
            preferred_element_type=jnp.float32,
        )

    return pl.pallas_call(
        body,
        grid=(pl.cdiv(VOCAB, VT),),
        in_specs=[
            pl.BlockSpec((D + 1, VT), lambda j: (0, j)),
            pl.BlockSpec((B, D + 1), lambda j: (0, 0)),
        ],
        out_specs=pl.BlockSpec((VT, B), lambda j: (j, 0)),
        out_shape=jax.ShapeDtypeStruct((VOCAB, B), jnp.float32),
    )(wt_aug, mean_aug)


def kernel(context_indices, embeddings, linear_w, linear_b):
    idx2 = context_indices.astype(jnp.int32).reshape(NW, IPW)
    emb2 = embeddings.reshape(VOCAB // 2, WIDE)
    mean = _mean_pool_sc(idx2, emb2).reshape(B, D)
    wt_aug = jnp.concatenate(
        [linear_w.T, linear_b[None, :]], axis=0).astype(jnp.bfloat16)
    mean_aug = jnp.concatenate(
        [mean, jnp.ones((B, 1), jnp.float32)], axis=1).astype(jnp.bfloat16)
    return _logits_tc(wt_aug, mean_aug).T


# speedup vs baseline: 1.0156x; 1.0156x over previous
import functools

import jax
import jax.numpy as jnp
from jax import lax
from jax.experimental import pallas as pl
from jax.experimental.pallas import tpu as pltpu
from jax.experimental.pallas import tpu_sc as plsc

VOCAB = 100000
D = 64
B = 4096
CTX = 20

NW = 32
BPW = B // NW
CHUNKS = 8
ROWS_PER_CHUNK = BPW // CHUNKS
IDX_PER_CHUNK = ROWS_PER_CHUNK * CTX
SUBS = ((0, 128), (128, 128), (256, 64))
LANES = 16

VT = 1024


def _mean_pool_sc(idx2, emb):
    mesh = plsc.VectorSubcoreMesh(core_axis_name="c", subcore_axis_name="s")

    @functools.partial(
        pl.kernel,
        mesh=mesh,
        compiler_params=pltpu.CompilerParams(use_tc_tiling_on_sc=False),
        out_type=jax.ShapeDtypeStruct((NW, BPW, D), jnp.float32),
        scratch_types=[
            pltpu.VMEM((BPW * CTX,), jnp.int32),
            pltpu.VMEM((2, IDX_PER_CHUNK, D), jnp.float32),
            pltpu.VMEM((BPW, D), jnp.float32),
            pltpu.SemaphoreType.DMA,
            pltpu.SemaphoreType.DMA,
        ],
    )
    def k(idx_hbm, emb_hbm, out_hbm, idx_v, rows_v, acc_v, sem0, sem1):
        wid = lax.axis_index("s") * 2 + lax.axis_index("c")
        sems = (sem0, sem1)
        pltpu.sync_copy(idx_hbm.at[wid], idx_v)

        def fire(c):
            return [
                pltpu.async_copy(
                    emb_hbm.at[idx_v.at[pl.ds(c * IDX_PER_CHUNK + o, n)]],
                    rows_v.at[c % 2, pl.ds(o, n)],
                    sems[c % 2],
                )
                for o, n in SUBS
            ]

        pending = fire(0)
        for c in range(CHUNKS):
            for h in pending:
                h.wait()
            if c + 1 < CHUNKS:
                pending = fire(c + 1)

            def body(r, carry, c=c):
                base = r * CTX
                buf = c % 2
                for g in range(D // LANES):
                    s = rows_v[buf, base, pl.ds(g * LANES, LANES)]
                    for kk in range(1, CTX):
                        s = s + rows_v[buf, base + kk, pl.ds(g * LANES, LANES)]
                    acc_v[c * ROWS_PER_CHUNK + r, pl.ds(g * LANES, LANES)] = (
                        s * (1.0 / CTX)
                    )
                return carry

            lax.fori_loop(0, ROWS_PER_CHUNK, body, 0)
        pltpu.sync_copy(acc_v, out_hbm.at[wid])

    return k(idx2, emb)


def _logits_tc(wt_aug, mean_aug):
    def body(w_ref, mean_ref, out_ref):
        out_ref[...] = lax.dot_general(
            w_ref[...], mean_ref[...], (((0,), (1,)), ((), ())),
            preferred_element_type=jnp.float32,
        )

    return pl.pallas_call(
        body,
        grid=(pl.cdiv(VOCAB, VT),),
        in_specs=[
            pl.BlockSpec((D + 1, VT), lambda j: (0, j)),
            pl.BlockSpec((B, D + 1), lambda j: (0, 0)),
        ],
        out_specs=pl.BlockSpec((VT, B), lambda j: (j, 0)),
        out_shape=jax.ShapeDtypeStruct((VOCAB, B), jnp.float32),
    )(wt_aug, mean_aug)


def kernel(context_indices, embeddings, linear_w, linear_b):
    idx2 = context_indices.astype(jnp.int32).reshape(NW, BPW * CTX)
    mean = _mean_pool_sc(idx2, embeddings).reshape(B, D)
    wt_aug = jnp.concatenate(
        [linear_w.T, linear_b[None, :]], axis=0).astype(jnp.bfloat16)
    mean_aug = jnp.concatenate(
        [mean, jnp.ones((B, 1), jnp.float32)], axis=1).astype(jnp.bfloat16)
    return _logits_tc(wt_aug, mean_aug).T
